# Initial kernel scaffold; baseline (speedup 1.0000x reference)
import functools

import jax
import jax.numpy as jnp
from jax import lax
from jax.experimental import pallas as pl
from jax.experimental.pallas import tpu as pltpu
from jax.experimental.pallas import tpu_sc as plsc

_NUM_CLASSES = 1024
_FEAT = 256
_BATCH = 16384
_ALPHA = 0.5
_THETA = 10.0

_L = 16
_NC = 2
_NS = 16
_NW = _NC * _NS
_BPW = _BATCH // _NW
_CK = 64
_NCH = _BPW // _CK
_FV = _FEAT // _L


def _sc_sumsq(x, labels, centers):
    mesh = plsc.VectorSubcoreMesh(core_axis_name="c", subcore_axis_name="s")

    @functools.partial(
        pl.kernel,
        out_type=jax.ShapeDtypeStruct((_BATCH,), jnp.float32),
        mesh=mesh,
        scratch_types=[
            pltpu.VMEM((_BPW,), jnp.int32),
            pltpu.VMEM((_CK, _FEAT), jnp.float32),
            pltpu.VMEM((_CK, _FEAT), jnp.float32),
            pltpu.VMEM((_CK, _FEAT), jnp.float32),
            pltpu.VMEM((_CK, _FEAT), jnp.float32),
            pltpu.VMEM((_CK,), jnp.float32),
            pltpu.SemaphoreType.DMA,
            pltpu.SemaphoreType.DMA,
            pltpu.SemaphoreType.DMA,
            pltpu.SemaphoreType.DMA,
        ],
    )
    def body(x_hbm, lbl_hbm, cen_hbm, out_hbm,
             lbl_v, xb0, xb1, cb0, cb1, ob, sx0, sx1, sc0, sc1):
        wid = lax.axis_index("s") * _NC + lax.axis_index("c")
        base = wid * _BPW
        pltpu.sync_copy(lbl_hbm.at[pl.ds(base, _BPW)], lbl_v)

        xbufs = (xb0, xb1)
        cbufs = (cb0, cb1)
        xsems = (sx0, sx1)
        csems = (sc0, sc1)

        def start(g, slot):
            row0 = base + g * _CK
            pltpu.async_copy(x_hbm.at[pl.ds(row0, _CK)], xbufs[slot], xsems[slot])
            pltpu.async_copy(cen_hbm.at[lbl_v.at[pl.ds(g * _CK, _CK)]],
                             cbufs[slot], csems[slot])

        def wait(g, slot):
            row0 = base + g * _CK
            pltpu.make_async_copy(x_hbm.at[pl.ds(row0, _CK)],
                                  xbufs[slot], xsems[slot]).wait()
            pltpu.make_async_copy(cen_hbm.at[lbl_v.at[pl.ds(g * _CK, _CK)]],
                                  cbufs[slot], csems[slot]).wait()

        start(0, 0)
        for g in range(_NCH):
            slot = g % 2
            if g + 1 < _NCH:
                start(g + 1, (g + 1) % 2)
            wait(g, slot)
            xb, cb = xbufs[slot], cbufs[slot]

            def row_body(r, carry, xb=xb, cb=cb):
                acc = jnp.zeros((_L,), jnp.float32)
                for f in range(_FV):
                    xv = xb[r, pl.ds(f * _L, _L)]
                    cv = cb[r, pl.ds(f * _L, _L)]
                    d = xv - cv
                    acc = acc + d * d
                ob[r] = jnp.sum(acc)
                return carry

            lax.fori_loop(0, _CK, row_body, 0)
            pltpu.sync_copy(ob, out_hbm.at[pl.ds(base + g * _CK, _CK)])

    return body(x, labels, centers)


def _tc_finalize(sq, a_col, b_row, d03):
    def body(sq_ref, a_ref, b_ref, d_ref, o_ref):
        dist = jnp.sqrt(sq_ref[...])
        loss = 0.5 * jnp.sum(dist) / _BATCH
        b = b_ref[...]

        def blk(i, tot):
            a = a_ref[pl.ds(i * 32, 32), :]
            t = _THETA - a + b
            return tot + jnp.sum(jnp.maximum(t, 0.0))

        rk1 = lax.fori_loop(0, _NUM_CLASSES // 32, blk, 0.0)
        rk2 = jnp.sum(jnp.maximum(2.0 * _THETA - d_ref[0, 0] + b, 0.0))
        o_ref[0, 0] = _ALPHA * (loss + rk1 + rk2)

    return pl.pallas_call(
        body,
        out_shape=jax.ShapeDtypeStruct((1, 1), jnp.float32),
    )(sq, a_col, b_row, d03)


def kernel(x, labels, centers, centers_distances):
    sumsq = _sc_sumsq(x, labels, centers)
    sq = sumsq.reshape(_BATCH // 128, 128)
    nc = _NUM_CLASSES
    d_i2 = centers_distances[: nc - 2, 2]
    d_j1 = centers_distances[: nc - 1, 1]
    a_col = jnp.pad(d_i2, (0, 2), constant_values=1e30).reshape(nc, 1)
    b_row = jnp.pad(d_j1, (0, 1), constant_values=-1e30).reshape(1, nc)
    d03 = centers_distances[0, 3].reshape(1, 1)
    out = _tc_finalize(sq, a_col, b_row, d03)
    return out[0, 0]


# speedup vs baseline: 1.4074x; 1.4074x over previous
import functools

import jax
import jax.numpy as jnp
from jax import lax
from jax.experimental import pallas as pl
from jax.experimental.pallas import tpu as pltpu
from jax.experimental.pallas import tpu_sc as plsc

_NUM_CLASSES = 1024
_FEAT = 256
_BATCH = 16384
_ALPHA = 0.5
_THETA = 10.0

_L = 16
_NC = 2
_NS = 16
_NW = _NC * _NS
_BPW = _BATCH // _NW
_CK = 64
_NCH = _BPW // _CK
_FV = _FEAT // _L


def _sc_sumsq(x, labels, centers):
    mesh = plsc.VectorSubcoreMesh(core_axis_name="c", subcore_axis_name="s")

    @functools.partial(
        pl.kernel,
        out_type=jax.ShapeDtypeStruct((_BATCH, _L), jnp.float32),
        mesh=mesh,
        scratch_types=[
            pltpu.VMEM((_BPW,), jnp.int32),
            pltpu.VMEM((_CK, _FEAT), jnp.float32),
            pltpu.VMEM((_CK, _FEAT), jnp.float32),
            pltpu.VMEM((_CK, _FEAT), jnp.float32),
            pltpu.VMEM((_CK, _FEAT), jnp.float32),
            pltpu.VMEM((_CK, _L), jnp.float32),
            pltpu.SemaphoreType.DMA,
            pltpu.SemaphoreType.DMA,
            pltpu.SemaphoreType.DMA,
            pltpu.SemaphoreType.DMA,
        ],
    )
    def body(x_hbm, lbl_hbm, cen_hbm, out_hbm,
             lbl_v, xb0, xb1, cb0, cb1, ob, sx0, sx1, sc0, sc1):
        wid = lax.axis_index("s") * _NC + lax.axis_index("c")
        base = wid * _BPW
        pltpu.sync_copy(lbl_hbm.at[pl.ds(base, _BPW)], lbl_v)

        xbufs = (xb0, xb1)
        cbufs = (cb0, cb1)
        xsems = (sx0, sx1)
        csems = (sc0, sc1)

        def start(g, slot):
            row0 = base + g * _CK
            pltpu.async_copy(x_hbm.at[pl.ds(row0, _CK)], xbufs[slot], xsems[slot])
            pltpu.async_copy(cen_hbm.at[lbl_v.at[pl.ds(g * _CK, _CK)]],
                             cbufs[slot], csems[slot])

        def wait(g, slot):
            row0 = base + g * _CK
            pltpu.make_async_copy(x_hbm.at[pl.ds(row0, _CK)],
                                  xbufs[slot], xsems[slot]).wait()
            pltpu.make_async_copy(cen_hbm.at[lbl_v.at[pl.ds(g * _CK, _CK)]],
                                  cbufs[slot], csems[slot]).wait()

        start(0, 0)
        for g in range(_NCH):
            slot = g % 2
            if g + 1 < _NCH:
                start(g + 1, (g + 1) % 2)
            wait(g, slot)
            xb, cb = xbufs[slot], cbufs[slot]

            def row_body(r, carry, xb=xb, cb=cb):
                acc = jnp.zeros((_L,), jnp.float32)
                for f in range(_FV):
                    xv = xb[r, pl.ds(f * _L, _L)]
                    cv = cb[r, pl.ds(f * _L, _L)]
                    d = xv - cv
                    acc = acc + d * d
                ob[r] = acc
                return carry

            lax.fori_loop(0, _CK, row_body, 0)
            pltpu.sync_copy(ob, out_hbm.at[pl.ds(base + g * _CK, _CK)])

    return body(x, labels, centers)


def _tc_finalize(sq, a_col, b_row, d03):
    def body(sq_ref, a_ref, b_ref, d_ref, o_ref):
        part = sq_ref[...]
        lane = lax.broadcasted_iota(jnp.int32, (128, 8), 0)
        grp = lax.broadcasted_iota(jnp.int32, (128, 8), 1)
        gmat = jnp.where(lane // _L == grp, 1.0, 0.0).astype(jnp.float32)
        sumsq = jnp.dot(part, gmat, preferred_element_type=jnp.float32)
        dist = jnp.sqrt(sumsq)
        loss = 0.5 * jnp.sum(dist) / _BATCH
        b = b_ref[...]

        def blk(i, tot):
            a = a_ref[pl.ds(i * 32, 32), :]
            t = _THETA - a + b
            return tot + jnp.sum(jnp.maximum(t, 0.0))

        rk1 = lax.fori_loop(0, _NUM_CLASSES // 32, blk, 0.0)
        rk2 = jnp.sum(jnp.maximum(2.0 * _THETA - d_ref[0, 0] + b, 0.0))
        o_ref[...] = (_ALPHA * (loss + rk1 + rk2)).reshape(1, 1)

    return pl.pallas_call(
        body,
        out_shape=jax.ShapeDtypeStruct((1, 1), jnp.float32),
    )(sq, a_col, b_row, d03)


def kernel(x, labels, centers, centers_distances):
    partials = _sc_sumsq(x, labels, centers)
    sq = partials.reshape(_BATCH * _L // 128, 128)
    nc = _NUM_CLASSES
    d_i2 = centers_distances[: nc - 2, 2]
    d_j1 = centers_distances[: nc - 1, 1]
    a_col = jnp.pad(d_i2, (0, 2), constant_values=1e30).reshape(nc, 1)
    b_row = jnp.pad(d_j1, (0, 1), constant_values=-1e30).reshape(1, nc)
    d03 = centers_distances[0, 3].reshape(1, 1)
    out = _tc_finalize(sq, a_col, b_row, d03)
    return out[0, 0]


# speedup vs baseline: 1.5290x; 1.0864x over previous
import functools

import jax
import jax.numpy as jnp
from jax import lax
from jax.experimental import pallas as pl
from jax.experimental.pallas import tpu as pltpu
from jax.experimental.pallas import tpu_sc as plsc

_NUM_CLASSES = 1024
_FEAT = 256
_BATCH = 16384
_ALPHA = 0.5
_THETA = 10.0

_L = 16
_NC = 2
_NS = 16
_NW = _NC * _NS
_BPW = _BATCH // _NW
_CK = 64
_NCH = _BPW // _CK
_FV = _FEAT // _L


def _sc_sumsq(x, labels, centers):
    mesh = plsc.VectorSubcoreMesh(core_axis_name="c", subcore_axis_name="s")

    @functools.partial(
        pl.kernel,
        out_type=jax.ShapeDtypeStruct((_BATCH, _L), jnp.float32),
        mesh=mesh,
        scratch_types=[
            pltpu.VMEM((_BPW,), jnp.int32),
            pltpu.VMEM((_CK, _FEAT), jnp.float32),
            pltpu.VMEM((_CK, _FEAT), jnp.float32),
            pltpu.VMEM((_CK, _FEAT), jnp.float32),
            pltpu.VMEM((_CK, _FEAT), jnp.float32),
            pltpu.VMEM((_CK, _L), jnp.float32),
            pltpu.SemaphoreType.DMA,
            pltpu.SemaphoreType.DMA,
            pltpu.SemaphoreType.DMA,
            pltpu.SemaphoreType.DMA,
        ],
    )
    def body(x_hbm, lbl_hbm, cen_hbm, out_hbm,
             lbl_v, xb0, xb1, cb0, cb1, ob, sx0, sx1, sc0, sc1):
        wid = lax.axis_index("s") * _NC + lax.axis_index("c")
        base = wid * _BPW
        pltpu.sync_copy(lbl_hbm.at[pl.ds(base, _BPW)], lbl_v)

        xbufs = (xb0, xb1)
        cbufs = (cb0, cb1)
        xsems = (sx0, sx1)
        csems = (sc0, sc1)

        def start(g, slot):
            row0 = base + g * _CK
            pltpu.async_copy(x_hbm.at[pl.ds(row0, _CK)], xbufs[slot], xsems[slot])
            pltpu.async_copy(cen_hbm.at[lbl_v.at[pl.ds(g * _CK, _CK)]],
                             cbufs[slot], csems[slot])

        def wait(g, slot):
            row0 = base + g * _CK
            pltpu.make_async_copy(x_hbm.at[pl.ds(row0, _CK)],
                                  xbufs[slot], xsems[slot]).wait()
            pltpu.make_async_copy(cen_hbm.at[lbl_v.at[pl.ds(g * _CK, _CK)]],
                                  cbufs[slot], csems[slot]).wait()

        start(0, 0)
        for g in range(_NCH):
            slot = g % 2
            if g + 1 < _NCH:
                start(g + 1, (g + 1) % 2)
            wait(g, slot)
            xb, cb = xbufs[slot], cbufs[slot]

            def row_body(r, carry, xb=xb, cb=cb):
                acc = jnp.zeros((_L,), jnp.float32)
                for f in range(_FV):
                    xv = xb[r, pl.ds(f * _L, _L)]
                    cv = cb[r, pl.ds(f * _L, _L)]
                    d = xv - cv
                    acc = acc + d * d
                ob[r] = acc
                return carry

            lax.fori_loop(0, _CK, row_body, 0)
            pltpu.sync_copy(ob, out_hbm.at[pl.ds(base + g * _CK, _CK)])

    return body(x, labels, centers)


def _tc_finalize(sq, a_col, bd):
    def body(sq_ref, a_ref, bd_ref, o_ref):
        part = sq_ref[...]
        lane = lax.broadcasted_iota(jnp.int32, (128, 8), 0)
        grp = lax.broadcasted_iota(jnp.int32, (128, 8), 1)
        gmat = jnp.where(lane // _L == grp, 1.0, 0.0).astype(jnp.float32)
        sumsq = jnp.dot(part, gmat, preferred_element_type=jnp.float32)
        dist = jnp.sqrt(sumsq)
        loss = 0.5 * jnp.sum(dist) / _BATCH
        b = bd_ref[0:1, :]
        d03 = bd_ref[1:2, 0:1]

        def blk(i, acc):
            a = a_ref[pl.ds(i * 32, 32), :]
            return acc + jnp.maximum(_THETA - a + b, 0.0)

        acc = lax.fori_loop(0, _NUM_CLASSES // 32, blk,
                            jnp.zeros((32, _NUM_CLASSES), jnp.float32))
        rk1 = jnp.sum(acc)
        rk2 = jnp.sum(jnp.maximum(2.0 * _THETA - d03 + b, 0.0))
        o_ref[...] = (_ALPHA * (loss + rk1 + rk2)).reshape(1, 1)

    return pl.pallas_call(
        body,
        out_shape=jax.ShapeDtypeStruct((1, 1), jnp.float32),
    )(sq, a_col, bd)


def kernel(x, labels, centers, centers_distances):
    partials = _sc_sumsq(x, labels, centers)
    sq = partials.reshape(_BATCH * _L // 128, 128)
    nc = _NUM_CLASSES
    d_i2 = centers_distances[: nc - 2, 2]
    d_j1 = centers_distances[: nc - 1, 1]
    a_col = jnp.pad(d_i2, (0, 2), constant_values=1e30).reshape(nc, 1)
    b_pad = jnp.pad(d_j1, (0, 1), constant_values=-1e30)
    bd = jnp.stack([b_pad, jnp.broadcast_to(centers_distances[0, 3], (nc,))])
    out = _tc_finalize(sq, a_col, bd)
    return out[0, 0]


# speedup vs baseline: 1.7149x; 1.1216x over previous
import functools

import jax
import jax.numpy as jnp
from jax import lax
from jax.experimental import pallas as pl
from jax.experimental.pallas import tpu as pltpu
from jax.experimental.pallas import tpu_sc as plsc

_NUM_CLASSES = 1024
_FEAT = 256
_BATCH = 16384
_ALPHA = 0.5
_THETA = 10.0

_L = 16
_NC = 2
_NS = 16
_NW = _NC * _NS
_BPW = _BATCH // _NW
_CK = 64
_NCH = _BPW // _CK
_FV = _FEAT // _L


def _sc_sumsq(x, labels, centers):
    mesh = plsc.VectorSubcoreMesh(core_axis_name="c", subcore_axis_name="s")

    @functools.partial(
        pl.kernel,
        out_type=jax.ShapeDtypeStruct((_BATCH,), jnp.float32),
        mesh=mesh,
        scratch_types=[
            pltpu.VMEM((_BPW,), jnp.int32),
            pltpu.VMEM((_CK, _FEAT), jnp.float32),
            pltpu.VMEM((_CK, _FEAT), jnp.float32),
            pltpu.VMEM((_CK, _FEAT), jnp.float32),
            pltpu.VMEM((_CK, _FEAT), jnp.float32),
            pltpu.VMEM((_CK,), jnp.float32),
            pltpu.SemaphoreType.DMA,
            pltpu.SemaphoreType.DMA,
            pltpu.SemaphoreType.DMA,
            pltpu.SemaphoreType.DMA,
        ],
    )
    def body(x_hbm, lbl_hbm, cen_hbm, out_hbm,
             lbl_v, xb0, xb1, cb0, cb1, ob, sx0, sx1, sc0, sc1):
        wid = lax.axis_index("s") * _NC + lax.axis_index("c")
        base = wid * _BPW
        pltpu.sync_copy(lbl_hbm.at[pl.ds(base, _BPW)], lbl_v)

        xbufs = (xb0, xb1)
        cbufs = (cb0, cb1)
        xsems = (sx0, sx1)
        csems = (sc0, sc1)

        def start(g, slot):
            row0 = base + g * _CK
            pltpu.async_copy(x_hbm.at[pl.ds(row0, _CK)], xbufs[slot], xsems[slot])
            pltpu.async_copy(cen_hbm.at[lbl_v.at[pl.ds(g * _CK, _CK)]],
                             cbufs[slot], csems[slot])

        def wait(g, slot):
            row0 = base + g * _CK
            pltpu.make_async_copy(x_hbm.at[pl.ds(row0, _CK)],
                                  xbufs[slot], xsems[slot]).wait()
            pltpu.make_async_copy(cen_hbm.at[lbl_v.at[pl.ds(g * _CK, _CK)]],
                                  cbufs[slot], csems[slot]).wait()

        start(0, 0)
        for g in range(_NCH):
            slot = g % 2
            if g + 1 < _NCH:
                start(g + 1, (g + 1) % 2)
            wait(g, slot)
            xb, cb = xbufs[slot], cbufs[slot]

            lanes = lax.iota(jnp.int32, _L)
            perms = [lanes ^ sh for sh in (8, 4, 2, 1)]

            def row_body(r, vec, xb=xb, cb=cb):
                acc = jnp.zeros((_L,), jnp.float32)
                for f in range(_FV):
                    xv = xb[r, pl.ds(f * _L, _L)]
                    cv = cb[r, pl.ds(f * _L, _L)]
                    d = xv - cv
                    acc = acc + d * d
                for perm in perms:
                    acc = acc + acc.at[perm].get(mode="promise_in_bounds")
                m = lanes == lax.broadcast_in_dim(r % _L, (_L,), ())
                vec = jnp.where(m, acc, vec)

                @pl.when(r % _L == _L - 1)
                def _():
                    ob[pl.ds((r // _L) * _L, _L)] = vec

                return vec

            lax.fori_loop(0, _CK, row_body, jnp.zeros((_L,), jnp.float32))
            pltpu.sync_copy(ob, out_hbm.at[pl.ds(base + g * _CK, _CK)])

    return body(x, labels, centers)


def _tc_finalize(sq, a_col, bd):
    def body(sq_ref, a_ref, bd_ref, o_ref):
        dist = jnp.sqrt(sq_ref[...])
        loss = 0.5 * jnp.sum(dist) / _BATCH
        b = bd_ref[0:1, :]
        d03 = bd_ref[1:2, 0:1]

        def blk(i, acc):
            a = a_ref[pl.ds(i * 32, 32), :]
            return acc + jnp.maximum(_THETA - a + b, 0.0)

        acc = lax.fori_loop(0, _NUM_CLASSES // 32, blk,
                            jnp.zeros((32, _NUM_CLASSES), jnp.float32))
        rk1 = jnp.sum(acc)
        rk2 = jnp.sum(jnp.maximum(2.0 * _THETA - d03 + b, 0.0))
        o_ref[...] = (_ALPHA * (loss + rk1 + rk2)).reshape(1, 1)

    return pl.pallas_call(
        body,
        out_shape=jax.ShapeDtypeStruct((1, 1), jnp.float32),
    )(sq, a_col, bd)


def kernel(x, labels, centers, centers_distances):
    sumsq = _sc_sumsq(x, labels, centers)
    sq = sumsq.reshape(_BATCH // 128, 128)
    nc = _NUM_CLASSES
    d_i2 = centers_distances[: nc - 2, 2]
    d_j1 = centers_distances[: nc - 1, 1]
    a_col = jnp.pad(d_i2, (0, 2), constant_values=1e30).reshape(nc, 1)
    b_pad = jnp.pad(d_j1, (0, 1), constant_values=-1e30)
    bd = jnp.stack([b_pad, jnp.broadcast_to(centers_distances[0, 3], (nc,))])
    out = _tc_finalize(sq, a_col, bd)
    return out[0, 0]
